# Initial kernel scaffold; baseline (speedup 1.0000x reference)
import jax
import jax.numpy as jnp
from jax.experimental import pallas as pl


def kernel(x, raw_weights, group):
    raise NotImplementedError("write your pallas kernel here")


# speedup vs baseline: 35.4581x; 35.4581x over previous
import jax
import jax.numpy as jnp
from jax.experimental import pallas as pl

_N_NET = 10


def _pool_kernel(w_ref, g_ref, x_ref, o_ref):
    w = w_ref[:, :]
    g = g_ref[:, :]
    n_roi = w.shape[1]
    row = jax.lax.broadcasted_iota(jnp.int32, (_N_NET, n_roi), 0)
    mask = g == row
    s_masked = jnp.where(mask, w, -jnp.inf)
    m = jnp.max(s_masked, axis=1, keepdims=True)
    m = jnp.where(jnp.isfinite(m), m, 0.0)
    e = jnp.where(mask, jnp.exp(w - m), 0.0)
    s = jnp.sum(e, axis=1, keepdims=True)
    b = e / jnp.where(s == 0.0, 1.0, s)
    o_ref[:, :] = jnp.dot(b, x_ref[:, :], preferred_element_type=jnp.float32)


def kernel(x, raw_weights, group):
    n_roi, feat = x.shape
    w2 = raw_weights.reshape(1, n_roi)
    g2 = group.reshape(1, n_roi).astype(jnp.int32)
    return pl.pallas_call(
        _pool_kernel,
        out_shape=jax.ShapeDtypeStruct((_N_NET, feat), jnp.float32),
    )(w2, g2, x)
